# baseline (speedup 1.0000x reference)
import jax
import jax.numpy as jnp
from jax import lax
from jax.experimental import pallas as pl
from jax.experimental.pallas import tpu as pltpu

_ORDER = 50
_EMBED = 64
_LANES = 128


def _body(idx_s, pos_s, table_t, out_v, rows_v, sem):
    copies = []
    for k in range(_ORDER):
        grp = jax.lax.shift_right_logical(jnp.maximum(idx_s[k], 0), 7)
        copies.append(pltpu.make_async_copy(
            table_t.at[:, pl.ds(grp * _LANES, _LANES)],
            rows_v.at[pl.ds(k * _EMBED, _EMBED), :], sem))
    for cp in copies:
        cp.start()

    lane = lax.broadcasted_iota(jnp.int32, (1, _LANES), 1)

    wis, rems = [], []
    denom = jnp.float32(0.0)
    for k in range(_ORDER):
        row = idx_s[k]
        wi = jnp.where(row != -1, pos_s[k], jnp.float32(0.0))
        denom = denom + wi
        wis.append(wi)
        rems.append(jnp.maximum(row, 0) & (_LANES - 1))

    acc = jnp.zeros((_EMBED, _LANES), jnp.float32)
    for k in range(_ORDER):
        copies[k].wait()
        block = rows_v[pl.ds(k * _EMBED, _EMBED), :]
        picked = jnp.where(lane == rems[k], block, jnp.float32(0.0))
        acc = acc + picked * wis[k]

    out_v[...] = jnp.sum(acc, axis=1, keepdims=True) / denom


@jax.jit
def kernel(partial_path_candidate, objects_embeds, positional_encoding):
    table_t = objects_embeds.T
    out = pl.pallas_call(
        _body,
        out_shape=jax.ShapeDtypeStruct((_EMBED, 1), jnp.float32),
        in_specs=[
            pl.BlockSpec(memory_space=pltpu.SMEM),
            pl.BlockSpec(memory_space=pltpu.SMEM),
            pl.BlockSpec(memory_space=pl.ANY),
        ],
        out_specs=pl.BlockSpec(memory_space=pltpu.VMEM),
        scratch_shapes=[
            pltpu.VMEM((_ORDER * _EMBED, _LANES), jnp.float32),
            pltpu.SemaphoreType.DMA,
        ],
    )(partial_path_candidate, positional_encoding, table_t)
    return out.reshape(_EMBED)
